# Initial kernel scaffold; baseline (speedup 1.0000x reference)
import functools

import jax
import jax.numpy as jnp
from jax import lax
from jax.experimental import pallas as pl
from jax.experimental.pallas import tpu as pltpu
from jax.experimental.pallas import tpu_sc as plsc

HEADS = 16
NUM_CODES = 1024
CODEBOOK_DIM = 64

_T = 512


def _kv_body(codes_ref, wk_ref, wv_ref, kt_ref, v_ref):
    c = codes_ref[0]
    kt_ref[0] = lax.dot_general(
        wk_ref[0], c, (((0,), (1,)), ((), ())),
        preferred_element_type=jnp.float32, precision=lax.Precision.HIGHEST)
    v_ref[0] = lax.dot_general(
        c, wv_ref[0], (((1,), (0,)), ((), ())),
        preferred_element_type=jnp.float32, precision=lax.Precision.HIGHEST)


def _argmax_body(x_ref, kt_ref, idx_ref):
    h = pl.program_id(1)
    logits = lax.dot_general(
        x_ref[0], kt_ref[0], (((0,), (0,)), ((), ())),
        preferred_element_type=jnp.float32, precision=lax.Precision.HIGHEST)
    m = jnp.max(logits, axis=1, keepdims=True)
    jidx = lax.broadcasted_iota(jnp.int32, logits.shape, 1)
    cand = jnp.where(logits == m, jidx, NUM_CODES)
    idx_ref[0, 0, :] = jnp.min(cand, axis=1) + h * NUM_CODES


def _make_gather(total_rows):
    info = plsc.get_sparse_core_info()
    nw = info.num_cores * info.num_subcores
    rows_per_w = total_rows // nw
    ch = 128
    n_ch = rows_per_w // ch
    mesh = plsc.VectorSubcoreMesh(core_axis_name="c", subcore_axis_name="s")

    @functools.partial(
        pl.kernel, mesh=mesh,
        out_type=jax.ShapeDtypeStruct((total_rows, CODEBOOK_DIM), jnp.float32),
        scratch_types=[
            pltpu.VMEM((ch,), jnp.int32),
            pltpu.VMEM((ch, CODEBOOK_DIM), jnp.float32),
            pltpu.SemaphoreType.DMA,
        ],
    )
    def gather(v_hbm, idx_hbm, out_hbm, idx_v, rows_v, sem):
        wid = lax.axis_index("s") * info.num_cores + lax.axis_index("c")
        base = wid * rows_per_w
        for ci in range(n_ch):
            off = base + ci * ch
            pltpu.sync_copy(idx_hbm.at[pl.ds(off, ch)], idx_v)
            pltpu.async_copy(v_hbm.at[idx_v], rows_v, sem).wait()
            pltpu.sync_copy(rows_v, out_hbm.at[pl.ds(off, ch)])

    return gather


def kernel(x, codes, Wk, Wv):
    b, c, n = x.shape
    nt = n // _T

    kt, v = pl.pallas_call(
        _kv_body,
        grid=(HEADS,),
        in_specs=[
            pl.BlockSpec((1, NUM_CODES, CODEBOOK_DIM), lambda h: (h, 0, 0)),
            pl.BlockSpec((1, CODEBOOK_DIM, CODEBOOK_DIM), lambda h: (h, 0, 0)),
            pl.BlockSpec((1, CODEBOOK_DIM, CODEBOOK_DIM), lambda h: (h, 0, 0)),
        ],
        out_specs=[
            pl.BlockSpec((1, CODEBOOK_DIM, NUM_CODES), lambda h: (h, 0, 0)),
            pl.BlockSpec((1, NUM_CODES, CODEBOOK_DIM), lambda h: (h, 0, 0)),
        ],
        out_shape=[
            jax.ShapeDtypeStruct((HEADS, CODEBOOK_DIM, NUM_CODES), jnp.float32),
            jax.ShapeDtypeStruct((HEADS, NUM_CODES, CODEBOOK_DIM), jnp.float32),
        ],
    )(codes, Wk, Wv)

    nb = b * HEADS * nt
    idx = pl.pallas_call(
        _argmax_body,
        grid=(b, HEADS, nt),
        in_specs=[
            pl.BlockSpec((1, CODEBOOK_DIM, _T), lambda bi, h, t: (bi, h, t)),
            pl.BlockSpec((1, CODEBOOK_DIM, NUM_CODES), lambda bi, h, t: (h, 0, 0)),
        ],
        out_specs=pl.BlockSpec(
            (1, 1, _T), lambda bi, h, t: ((bi * HEADS + h) * nt + t, 0, 0)),
        out_shape=jax.ShapeDtypeStruct((nb, 1, _T), jnp.int32),
    )(x, kt)

    total_rows = b * HEADS * n
    out_flat = _make_gather(total_rows)(
        v.reshape(HEADS * NUM_CODES, CODEBOOK_DIM), idx.reshape(total_rows))
    out = out_flat.reshape(b, HEADS, n, CODEBOOK_DIM)
    return out.transpose(0, 1, 3, 2).reshape(b, c, n)


# speedup vs baseline: 1.3681x; 1.3681x over previous
import functools

import jax
import jax.numpy as jnp
from jax import lax
from jax.experimental import pallas as pl
from jax.experimental.pallas import tpu as pltpu
from jax.experimental.pallas import tpu_sc as plsc

HEADS = 16
NUM_CODES = 1024
CODEBOOK_DIM = 64

_T = 512


def _kv_body(codes_ref, wk_ref, wv_ref, kt_ref, v_ref):
    c = codes_ref[0]
    kt_ref[0] = lax.dot_general(
        wk_ref[0], c, (((0,), (1,)), ((), ())),
        preferred_element_type=jnp.float32)
    v_ref[0] = lax.dot_general(
        c, wv_ref[0], (((1,), (0,)), ((), ())),
        preferred_element_type=jnp.float32)


def _argmax_body(x_ref, kt_ref, idx_ref):
    h = pl.program_id(1)
    logits = lax.dot_general(
        x_ref[0], kt_ref[0], (((0,), (0,)), ((), ())),
        preferred_element_type=jnp.float32)
    m = jnp.max(logits, axis=1, keepdims=True)
    jidx = lax.broadcasted_iota(jnp.int32, logits.shape, 1)
    cand = jnp.where(logits == m, jidx, NUM_CODES)
    idx_ref[0, 0, :] = jnp.min(cand, axis=1) + h * NUM_CODES


def _make_gather(total_rows):
    info = plsc.get_sparse_core_info()
    nw = info.num_cores * info.num_subcores
    rows_per_w = total_rows // nw
    ch = 128
    n_ch = rows_per_w // ch
    mesh = plsc.VectorSubcoreMesh(core_axis_name="c", subcore_axis_name="s")

    @functools.partial(
        pl.kernel, mesh=mesh,
        out_type=jax.ShapeDtypeStruct((total_rows, CODEBOOK_DIM), jnp.float32),
        scratch_types=[
            pltpu.VMEM((ch,), jnp.int32),
            pltpu.VMEM((ch, CODEBOOK_DIM), jnp.float32),
            pltpu.SemaphoreType.DMA,
        ],
        compiler_params=pltpu.CompilerParams(use_tc_tiling_on_sc=False),
    )
    def gather(v_hbm, idx_hbm, out_hbm, idx_v, rows_v, sem):
        wid = lax.axis_index("s") * info.num_cores + lax.axis_index("c")
        base = wid * rows_per_w
        for ci in range(n_ch):
            off = base + ci * ch
            pltpu.sync_copy(idx_hbm.at[pl.ds(off, ch)], idx_v)
            pltpu.async_copy(v_hbm.at[idx_v], rows_v, sem).wait()
            pltpu.sync_copy(rows_v, out_hbm.at[pl.ds(off, ch)])

    return gather


def kernel(x, codes, Wk, Wv):
    b, c, n = x.shape
    nt = n // _T

    kt, v = pl.pallas_call(
        _kv_body,
        grid=(HEADS,),
        in_specs=[
            pl.BlockSpec((1, NUM_CODES, CODEBOOK_DIM), lambda h: (h, 0, 0)),
            pl.BlockSpec((1, CODEBOOK_DIM, CODEBOOK_DIM), lambda h: (h, 0, 0)),
            pl.BlockSpec((1, CODEBOOK_DIM, CODEBOOK_DIM), lambda h: (h, 0, 0)),
        ],
        out_specs=[
            pl.BlockSpec((1, CODEBOOK_DIM, NUM_CODES), lambda h: (h, 0, 0)),
            pl.BlockSpec((1, NUM_CODES, CODEBOOK_DIM), lambda h: (h, 0, 0)),
        ],
        out_shape=[
            jax.ShapeDtypeStruct((HEADS, CODEBOOK_DIM, NUM_CODES), jnp.float32),
            jax.ShapeDtypeStruct((HEADS, NUM_CODES, CODEBOOK_DIM), jnp.float32),
        ],
    )(codes, Wk, Wv)

    nb = b * HEADS * nt
    idx = pl.pallas_call(
        _argmax_body,
        grid=(b, HEADS, nt),
        in_specs=[
            pl.BlockSpec((1, CODEBOOK_DIM, _T), lambda bi, h, t: (bi, h, t)),
            pl.BlockSpec((1, CODEBOOK_DIM, NUM_CODES), lambda bi, h, t: (h, 0, 0)),
        ],
        out_specs=pl.BlockSpec(
            (1, 1, _T), lambda bi, h, t: ((bi * HEADS + h) * nt + t, 0, 0)),
        out_shape=jax.ShapeDtypeStruct((nb, 1, _T), jnp.int32),
    )(x, kt)

    total_rows = b * HEADS * n
    out_flat = _make_gather(total_rows)(
        v.reshape(HEADS * NUM_CODES, CODEBOOK_DIM), idx.reshape(total_rows))
    out = out_flat.reshape(b, HEADS, n, CODEBOOK_DIM)
    return out.transpose(0, 1, 3, 2).reshape(b, c, n)


# speedup vs baseline: 2.2801x; 1.6666x over previous
import functools

import jax
import jax.numpy as jnp
from jax import lax
from jax.experimental import pallas as pl
from jax.experimental.pallas import tpu as pltpu
from jax.experimental.pallas import tpu_sc as plsc

HEADS = 16
NUM_CODES = 1024
CODEBOOK_DIM = 64


def _fused_body(codes_ref, wk_ref, wv_ref, x_ref, idx_ref, vt_ref, kt_ref):
    bi = pl.program_id(1)

    @pl.when(bi == 0)
    def _project():
        c = codes_ref[0]
        kt_ref[...] = lax.dot_general(
            wk_ref[0], c, (((0,), (1,)), ((), ())),
            preferred_element_type=jnp.float32)
        vt_ref[0] = lax.dot_general(
            wv_ref[0], c, (((0,), (1,)), ((), ())),
            preferred_element_type=jnp.float32)

    logits = lax.dot_general(
        kt_ref[...], x_ref[0], (((0,), (0,)), ((), ())),
        preferred_element_type=jnp.float32)
    m = jnp.max(logits, axis=0, keepdims=True)
    jidx = lax.broadcasted_iota(jnp.int32, logits.shape, 0)
    cand = jnp.where(logits == m, jidx, NUM_CODES)
    idx_ref[0, 0, :] = jnp.min(cand, axis=0)


def _make_gather_t(b, n):
    info = plsc.get_sparse_core_info()
    nw = info.num_cores * info.num_subcores
    assert b * HEADS == nw
    chunk = 512
    n_ch = n // chunk
    groups = chunk // 16
    mesh = plsc.VectorSubcoreMesh(core_axis_name="c", subcore_axis_name="s")

    @functools.partial(
        pl.kernel, mesh=mesh,
        out_type=jax.ShapeDtypeStruct((b, HEADS * CODEBOOK_DIM, n), jnp.float32),
        scratch_types=[
            pltpu.VMEM((n,), jnp.int32),
            pltpu.VMEM((CODEBOOK_DIM, NUM_CODES), jnp.float32),
            pltpu.VMEM((CODEBOOK_DIM, chunk), jnp.float32),
        ],
        compiler_params=pltpu.CompilerParams(
            use_tc_tiling_on_sc=False, needs_layout_passes=False),
    )
    def gather(vt_hbm, idx_hbm, out_hbm, idx_v, vt_v, rows_v):
        w = lax.axis_index("s") * info.num_cores + lax.axis_index("c")
        bi = w // HEADS
        h = w % HEADS
        pltpu.sync_copy(idx_hbm.at[w], idx_v)
        pltpu.sync_copy(vt_hbm.at[h], vt_v)
        for ci in range(n_ch):
            def body(g, _, _ci=ci):
                tvec = idx_v[pl.ds(_ci * chunk + g * 16, 16)]
                for dd in range(CODEBOOK_DIM):
                    dvec = jnp.full((16,), dd, jnp.int32)
                    rows_v[dd, pl.ds(g * 16, 16)] = plsc.load_gather(
                        vt_v, [dvec, tvec])
                return 0

            lax.fori_loop(0, groups, body, 0)
            pltpu.sync_copy(
                rows_v,
                out_hbm.at[bi, pl.ds(h * CODEBOOK_DIM, CODEBOOK_DIM),
                           pl.ds(ci * chunk, chunk)])

    return gather


def kernel(x, codes, Wk, Wv):
    b, c, n = x.shape

    idx, vt = pl.pallas_call(
        _fused_body,
        grid=(HEADS, b),
        in_specs=[
            pl.BlockSpec((1, NUM_CODES, CODEBOOK_DIM), lambda h, bi: (h, 0, 0)),
            pl.BlockSpec((1, CODEBOOK_DIM, CODEBOOK_DIM), lambda h, bi: (h, 0, 0)),
            pl.BlockSpec((1, CODEBOOK_DIM, CODEBOOK_DIM), lambda h, bi: (h, 0, 0)),
            pl.BlockSpec((1, CODEBOOK_DIM, n), lambda h, bi: (bi, h, 0)),
        ],
        out_specs=[
            pl.BlockSpec((1, 1, n), lambda h, bi: (bi * HEADS + h, 0, 0)),
            pl.BlockSpec((1, CODEBOOK_DIM, NUM_CODES), lambda h, bi: (h, 0, 0)),
        ],
        out_shape=[
            jax.ShapeDtypeStruct((b * HEADS, 1, n), jnp.int32),
            jax.ShapeDtypeStruct((HEADS, CODEBOOK_DIM, NUM_CODES), jnp.float32),
        ],
        scratch_shapes=[pltpu.VMEM((CODEBOOK_DIM, NUM_CODES), jnp.float32)],
    )(codes, Wk, Wv, x)

    return _make_gather_t(b, n)(vt, idx.reshape(b * HEADS, n))
